# baseline (speedup 1.0000x reference)
import functools

import jax
import jax.numpy as jnp
from jax import lax
from jax.experimental import pallas as pl
from jax.experimental.pallas import tpu as pltpu
from jax.experimental.pallas import tpu_sc as plsc

NC = 2
NS = 16
L = 16
TILES = NC * NS
EB = 128
CH = 16
RB = 512


def _sc_mesh():
    return plsc.VectorSubcoreMesh(core_axis_name="c", subcore_axis_name="s")


_SC_PARAMS = pltpu.CompilerParams(use_tc_tiling_on_sc=False)


@functools.lru_cache(maxsize=None)
def _make_deg(npad, nb):
    rpt = npad // NS

    @functools.partial(
        pl.kernel,
        out_type=jax.ShapeDtypeStruct((NC, npad), jnp.float32),
        mesh=_sc_mesh(),
        compiler_params=_SC_PARAMS,
        scratch_types=[
            pltpu.VMEM((nb, EB), jnp.int32),
            pltpu.VMEM((EB,), jnp.float32),
            pltpu.VMEM((rpt,), jnp.float32),
            pltpu.VMEM_SHARED((npad,), jnp.float32),
        ],
    )
    def deg_kernel(dst_hbm, out_hbm, dst_v, ones_v, zer_v, acc):
        c = lax.axis_index("c")
        s = lax.axis_index("s")
        w = c * NS + s

        @pl.loop(0, EB // L)
        def _(i):
            ones_v[pl.ds(i * L, L)] = jnp.ones((L,), jnp.float32)

        @pl.loop(0, rpt // L)
        def _(i):
            zer_v[pl.ds(i * L, L)] = jnp.zeros((L,), jnp.float32)

        pltpu.sync_copy(zer_v, acc.at[pl.ds(s * rpt, rpt)])
        plsc.subcore_barrier()

        pltpu.sync_copy(dst_hbm.at[w], dst_v)

        @pl.loop(0, nb)
        def _(j):
            pltpu.sync_copy(ones_v, acc.at[dst_v.at[j]], add=True)

        plsc.subcore_barrier()
        pltpu.sync_copy(acc.at[pl.ds(s * rpt, rpt)],
                        out_hbm.at[c, pl.ds(s * rpt, rpt)])

    return deg_kernel


@functools.lru_cache(maxsize=None)
def _make_prop(npad, nb, wd):
    rpt = npad // NS
    nch = nb // CH

    @functools.partial(
        pl.kernel,
        out_type=jax.ShapeDtypeStruct((NC, npad, wd), jnp.float32),
        mesh=_sc_mesh(),
        compiler_params=_SC_PARAMS,
        scratch_types=[
            pltpu.VMEM((2, CH, EB), jnp.int32),
            pltpu.VMEM((2, CH, EB), jnp.int32),
            pltpu.VMEM((2, EB, wd), jnp.float32),
            pltpu.VMEM_SHARED((npad, wd), jnp.float32),
            pltpu.SemaphoreType.DMA,
            pltpu.SemaphoreType.DMA,
            pltpu.SemaphoreType.DMA,
        ],
    )
    def prop_kernel(y_hbm, src_hbm, dst_hbm, out_hbm,
                    src_v, dst_v, rows_v, acc, sem_i, sem_g, sem_s):
        c = lax.axis_index("c")
        s = lax.axis_index("s")
        w = c * NS + s

        @pl.loop(0, EB)
        def _(r):
            @pl.loop(0, wd // L)
            def _(q):
                rows_v[0, r, pl.ds(q * L, L)] = jnp.zeros((L,), jnp.float32)

        @pl.loop(0, rpt // EB)
        def _(i):
            pltpu.sync_copy(rows_v.at[0], acc.at[pl.ds(s * rpt + i * EB, EB)])

        plsc.subcore_barrier()

        pltpu.sync_copy(src_hbm.at[w, pl.ds(0, CH)], src_v.at[0])
        pltpu.sync_copy(dst_hbm.at[w, pl.ds(0, CH)], dst_v.at[0])
        pltpu.async_copy(y_hbm.at[src_v.at[0, 0]], rows_v.at[0], sem_g)

        @pl.loop(0, nb)
        def _(t):
            m = lax.div(t, CH)
            slot = lax.rem(t, CH)
            b = lax.rem(t, 2)

            @pl.when(t >= 1)
            def _():
                pltpu.make_async_copy(
                    y_hbm.at[pl.ds(0, EB)], rows_v.at[0], sem_s).wait()

            @pl.when((slot == 0) & (t + CH < nb))
            def _():
                mb = lax.rem(m + 1, 2)
                pltpu.async_copy(
                    src_hbm.at[w, pl.ds((m + 1) * CH, CH)],
                    src_v.at[mb], sem_i)
                pltpu.async_copy(
                    dst_hbm.at[w, pl.ds((m + 1) * CH, CH)],
                    dst_v.at[mb], sem_i)

            @pl.when((slot == CH - 1) & (t + 1 < nb))
            def _():
                for _ in range(2):
                    pltpu.make_async_copy(
                        src_hbm.at[w, pl.ds(0, CH)], src_v.at[0],
                        sem_i).wait()

            @pl.when(t + 1 < nb)
            def _():
                t1 = t + 1
                pltpu.async_copy(
                    y_hbm.at[src_v.at[lax.rem(lax.div(t1, CH), 2),
                                      lax.rem(t1, CH)]],
                    rows_v.at[lax.rem(t1, 2)], sem_g)

            pltpu.make_async_copy(
                y_hbm.at[src_v.at[lax.rem(m, 2), slot]],
                rows_v.at[b], sem_g).wait()
            pltpu.async_copy(rows_v.at[b],
                             acc.at[dst_v.at[lax.rem(m, 2), slot]],
                             sem_s, add=True)

        pltpu.make_async_copy(
            y_hbm.at[pl.ds(0, EB)], rows_v.at[0], sem_s).wait()

        plsc.subcore_barrier()
        pltpu.sync_copy(acc.at[pl.ds(s * rpt, rpt)],
                        out_hbm.at[c, pl.ds(s * rpt, rpt)])

    return prop_kernel


def _tc1(degp, xp, W1):
    npad = xp.shape[0]

    def body(deg_ref, x_ref, w1_ref, y_ref):
        dinv = lax.rsqrt(deg_ref[0] + deg_ref[1] + 1.0)
        y_ref[...] = dinv[:, None] * jnp.dot(
            x_ref[...], w1_ref[...], preferred_element_type=jnp.float32)

    return pl.pallas_call(
        body,
        grid=(npad // RB,),
        in_specs=[
            pl.BlockSpec((NC, RB), lambda i: (0, i)),
            pl.BlockSpec((RB, 128), lambda i: (i, 0)),
            pl.BlockSpec((128, 128), lambda i: (0, 0)),
        ],
        out_specs=pl.BlockSpec((RB, 128), lambda i: (i, 0)),
        out_shape=jax.ShapeDtypeStruct((npad, 128), jnp.float32),
    )(degp, xp, W1)


def _tc2(degp, s1, y1, b1, W2):
    npad = y1.shape[0]
    cd = W2.shape[1]

    def body(deg_ref, s1_ref, y1_ref, b1_ref, w2_ref, y2_ref):
        dinv = lax.rsqrt(deg_ref[0] + deg_ref[1] + 1.0)
        t = s1_ref[0] + s1_ref[1] + y1_ref[...]
        h = jnp.maximum(dinv[:, None] * t + b1_ref[...], 0.0)
        y2_ref[...] = dinv[:, None] * jnp.dot(
            h, w2_ref[...], preferred_element_type=jnp.float32)

    return pl.pallas_call(
        body,
        grid=(npad // RB,),
        in_specs=[
            pl.BlockSpec((NC, RB), lambda i: (0, i)),
            pl.BlockSpec((NC, RB, 128), lambda i: (0, i, 0)),
            pl.BlockSpec((RB, 128), lambda i: (i, 0)),
            pl.BlockSpec((128,), lambda i: (0,)),
            pl.BlockSpec((128, cd), lambda i: (0, 0)),
        ],
        out_specs=pl.BlockSpec((RB, cd), lambda i: (i, 0)),
        out_shape=jax.ShapeDtypeStruct((npad, cd), jnp.float32),
    )(degp, s1, y1, b1, W2)


def _tc3(degp, s2, y2, b2):
    npad = y2.shape[0]
    cd = y2.shape[1]

    def body(deg_ref, s2_ref, y2_ref, b2_ref, o_ref):
        dinv = lax.rsqrt(deg_ref[0] + deg_ref[1] + 1.0)
        o = dinv[:, None] * (s2_ref[0] + s2_ref[1] + y2_ref[...]) + b2_ref[...]
        m = jnp.max(o, axis=-1, keepdims=True)
        lse = jnp.log(jnp.sum(jnp.exp(o - m), axis=-1, keepdims=True)) + m
        o_ref[...] = o - lse

    return pl.pallas_call(
        body,
        grid=(npad // RB,),
        in_specs=[
            pl.BlockSpec((NC, RB), lambda i: (0, i)),
            pl.BlockSpec((NC, RB, cd), lambda i: (0, i, 0)),
            pl.BlockSpec((RB, cd), lambda i: (i, 0)),
            pl.BlockSpec((cd,), lambda i: (0,)),
        ],
        out_specs=pl.BlockSpec((RB, cd), lambda i: (i, 0)),
        out_shape=jax.ShapeDtypeStruct((npad, cd), jnp.float32),
    )(degp, s2, y2, b2)


def kernel(x, edge_index, W1, b1, W2, b2):
    n, d = x.shape
    e = edge_index.shape[1]

    npad = ((n + RB) // RB) * RB
    step = TILES * EB * CH
    epad = ((e + step - 1) // step) * step
    nb = epad // (TILES * EB)

    ei = edge_index.astype(jnp.int32)
    pad = jnp.full((epad - e,), n, jnp.int32)
    src_p = jnp.concatenate([ei[0], pad]).reshape(TILES, nb, EB)
    dst_p = jnp.concatenate([ei[1], pad]).reshape(TILES, nb, EB)
    xp = jnp.pad(x, ((0, npad - n), (0, 0)))

    degp = _make_deg(npad, nb)(dst_p)
    y1 = _tc1(degp, xp, W1)
    s1 = _make_prop(npad, nb, 128)(y1, src_p, dst_p)
    y2 = _tc2(degp, s1, y1, b1, W2)
    s2 = _make_prop(npad, nb, 16)(y2, src_p, dst_p)
    out = _tc3(degp, s2, y2, b2)
    return out[:n]


# speedup vs baseline: 1.0245x; 1.0245x over previous
import functools

import jax
import jax.numpy as jnp
from jax import lax
from jax.experimental import pallas as pl
from jax.experimental.pallas import tpu as pltpu
from jax.experimental.pallas import tpu_sc as plsc

NC = 2
NS = 16
L = 16
TILES = NC * NS
EB = 128
CH = 16
RB = 512


def _sc_mesh():
    return plsc.VectorSubcoreMesh(core_axis_name="c", subcore_axis_name="s")


_SC_PARAMS = pltpu.CompilerParams(use_tc_tiling_on_sc=False)


@functools.lru_cache(maxsize=None)
def _make_deg(npad, nb):
    rpt = npad // NS

    @functools.partial(
        pl.kernel,
        out_type=jax.ShapeDtypeStruct((NC, npad), jnp.float32),
        mesh=_sc_mesh(),
        compiler_params=_SC_PARAMS,
        scratch_types=[
            pltpu.VMEM((nb, EB), jnp.int32),
            pltpu.VMEM((EB,), jnp.float32),
            pltpu.VMEM((rpt,), jnp.float32),
            pltpu.VMEM_SHARED((npad,), jnp.float32),
        ],
    )
    def deg_kernel(dst_hbm, out_hbm, dst_v, ones_v, zer_v, acc):
        c = lax.axis_index("c")
        s = lax.axis_index("s")
        w = c * NS + s

        @pl.loop(0, EB // L)
        def _(i):
            ones_v[pl.ds(i * L, L)] = jnp.ones((L,), jnp.float32)

        @pl.loop(0, rpt // L)
        def _(i):
            zer_v[pl.ds(i * L, L)] = jnp.zeros((L,), jnp.float32)

        pltpu.sync_copy(zer_v, acc.at[pl.ds(s * rpt, rpt)])
        plsc.subcore_barrier()

        pltpu.sync_copy(dst_hbm.at[w], dst_v)

        @pl.loop(0, nb)
        def _(j):
            pltpu.sync_copy(ones_v, acc.at[dst_v.at[j]], add=True)

        plsc.subcore_barrier()
        pltpu.sync_copy(acc.at[pl.ds(s * rpt, rpt)],
                        out_hbm.at[c, pl.ds(s * rpt, rpt)])

    return deg_kernel


@functools.lru_cache(maxsize=None)
def _make_prop(npad, nb, wd):
    rpt = npad // NS

    @functools.partial(
        pl.kernel,
        out_type=jax.ShapeDtypeStruct((NC, npad, wd), jnp.float32),
        mesh=_sc_mesh(),
        compiler_params=_SC_PARAMS,
        scratch_types=[
            pltpu.VMEM((2, CH, EB), jnp.int32),
            pltpu.VMEM((2, CH, EB), jnp.int32),
            pltpu.VMEM((2, EB, wd), jnp.float32),
            pltpu.VMEM_SHARED((npad, wd), jnp.float32),
            pltpu.SemaphoreType.DMA,
        ],
    )
    def prop_kernel(y_hbm, src_hbm, dst_hbm, out_hbm,
                    src_v, dst_v, rows_v, acc, sem_g):
        c = lax.axis_index("c")
        s = lax.axis_index("s")
        w = c * NS + s

        @pl.loop(0, EB)
        def _(r):
            @pl.loop(0, wd // L)
            def _(q):
                rows_v[0, r, pl.ds(q * L, L)] = jnp.zeros((L,), jnp.float32)

        @pl.loop(0, rpt // EB)
        def _(i):
            pltpu.sync_copy(rows_v.at[0], acc.at[pl.ds(s * rpt + i * EB, EB)])

        plsc.subcore_barrier()

        pltpu.sync_copy(src_hbm.at[w, pl.ds(0, CH)], src_v.at[0])
        pltpu.sync_copy(dst_hbm.at[w, pl.ds(0, CH)], dst_v.at[0])
        pltpu.async_copy(y_hbm.at[src_v.at[0, 0]], rows_v.at[0], sem_g)

        @pl.loop(0, nb)
        def _(t):
            m = lax.div(t, CH)
            mb = lax.rem(m, 2)
            slot = lax.rem(t, CH)
            b = lax.rem(t, 2)

            pltpu.make_async_copy(
                y_hbm.at[src_v.at[mb, slot]], rows_v.at[b], sem_g).wait()

            @pl.when((slot == CH - 1) & (t + 1 < nb))
            def _():
                mb1 = lax.rem(m + 1, 2)
                pltpu.sync_copy(src_hbm.at[w, pl.ds((m + 1) * CH, CH)],
                                src_v.at[mb1])
                pltpu.sync_copy(dst_hbm.at[w, pl.ds((m + 1) * CH, CH)],
                                dst_v.at[mb1])

            @pl.when(t + 1 < nb)
            def _():
                t1 = t + 1
                pltpu.async_copy(
                    y_hbm.at[src_v.at[lax.rem(lax.div(t1, CH), 2),
                                      lax.rem(t1, CH)]],
                    rows_v.at[lax.rem(t1, 2)], sem_g)

            pltpu.sync_copy(rows_v.at[b], acc.at[dst_v.at[mb, slot]],
                            add=True)

        plsc.subcore_barrier()
        pltpu.sync_copy(acc.at[pl.ds(s * rpt, rpt)],
                        out_hbm.at[c, pl.ds(s * rpt, rpt)])

    return prop_kernel


def _tc1(degp, xp, W1):
    npad = xp.shape[0]

    def body(deg_ref, x_ref, w1_ref, y_ref):
        dinv = lax.rsqrt(deg_ref[0] + deg_ref[1] + 1.0)
        y_ref[...] = dinv[:, None] * jnp.dot(
            x_ref[...], w1_ref[...], preferred_element_type=jnp.float32)

    return pl.pallas_call(
        body,
        grid=(npad // RB,),
        in_specs=[
            pl.BlockSpec((NC, RB), lambda i: (0, i)),
            pl.BlockSpec((RB, 128), lambda i: (i, 0)),
            pl.BlockSpec((128, 128), lambda i: (0, 0)),
        ],
        out_specs=pl.BlockSpec((RB, 128), lambda i: (i, 0)),
        out_shape=jax.ShapeDtypeStruct((npad, 128), jnp.float32),
    )(degp, xp, W1)


def _tc2(degp, s1, y1, b1, W2):
    npad = y1.shape[0]
    cd = W2.shape[1]

    def body(deg_ref, s1_ref, y1_ref, b1_ref, w2_ref, y2_ref):
        dinv = lax.rsqrt(deg_ref[0] + deg_ref[1] + 1.0)
        t = s1_ref[0] + s1_ref[1] + y1_ref[...]
        h = jnp.maximum(dinv[:, None] * t + b1_ref[...], 0.0)
        y2_ref[...] = dinv[:, None] * jnp.dot(
            h, w2_ref[...], preferred_element_type=jnp.float32)

    return pl.pallas_call(
        body,
        grid=(npad // RB,),
        in_specs=[
            pl.BlockSpec((NC, RB), lambda i: (0, i)),
            pl.BlockSpec((NC, RB, 128), lambda i: (0, i, 0)),
            pl.BlockSpec((RB, 128), lambda i: (i, 0)),
            pl.BlockSpec((128,), lambda i: (0,)),
            pl.BlockSpec((128, cd), lambda i: (0, 0)),
        ],
        out_specs=pl.BlockSpec((RB, cd), lambda i: (i, 0)),
        out_shape=jax.ShapeDtypeStruct((npad, cd), jnp.float32),
    )(degp, s1, y1, b1, W2)


def _tc3(degp, s2, y2, b2):
    npad = y2.shape[0]
    cd = y2.shape[1]

    def body(deg_ref, s2_ref, y2_ref, b2_ref, o_ref):
        dinv = lax.rsqrt(deg_ref[0] + deg_ref[1] + 1.0)
        o = dinv[:, None] * (s2_ref[0] + s2_ref[1] + y2_ref[...]) + b2_ref[...]
        m = jnp.max(o, axis=-1, keepdims=True)
        lse = jnp.log(jnp.sum(jnp.exp(o - m), axis=-1, keepdims=True)) + m
        o_ref[...] = o - lse

    return pl.pallas_call(
        body,
        grid=(npad // RB,),
        in_specs=[
            pl.BlockSpec((NC, RB), lambda i: (0, i)),
            pl.BlockSpec((NC, RB, cd), lambda i: (0, i, 0)),
            pl.BlockSpec((RB, cd), lambda i: (i, 0)),
            pl.BlockSpec((cd,), lambda i: (0,)),
        ],
        out_specs=pl.BlockSpec((RB, cd), lambda i: (i, 0)),
        out_shape=jax.ShapeDtypeStruct((npad, cd), jnp.float32),
    )(degp, s2, y2, b2)


def kernel(x, edge_index, W1, b1, W2, b2):
    n, d = x.shape
    e = edge_index.shape[1]

    npad = ((n + RB) // RB) * RB
    step = TILES * EB * CH
    epad = ((e + step - 1) // step) * step
    nb = epad // (TILES * EB)

    ei = edge_index.astype(jnp.int32)
    pad = jnp.full((epad - e,), n, jnp.int32)
    src_p = jnp.concatenate([ei[0], pad]).reshape(TILES, nb, EB)
    dst_p = jnp.concatenate([ei[1], pad]).reshape(TILES, nb, EB)
    xp = jnp.pad(x, ((0, npad - n), (0, 0)))

    degp = _make_deg(npad, nb)(dst_p)
    y1 = _tc1(degp, xp, W1)
    s1 = _make_prop(npad, nb, 128)(y1, src_p, dst_p)
    y2 = _tc2(degp, s1, y1, b1, W2)
    s2 = _make_prop(npad, nb, 16)(y2, src_p, dst_p)
    out = _tc3(degp, s2, y2, b2)
    return out[:n]


# speedup vs baseline: 1.7472x; 1.7055x over previous
import functools

import jax
import jax.numpy as jnp
from jax import lax
from jax.experimental import pallas as pl
from jax.experimental.pallas import tpu as pltpu
from jax.experimental.pallas import tpu_sc as plsc

NC = 2
NS = 16
L = 16
TILES = NC * NS
EB = 128
CH = 16
RB = 512


def _sc_mesh():
    return plsc.VectorSubcoreMesh(core_axis_name="c", subcore_axis_name="s")


_SC_PARAMS = pltpu.CompilerParams(use_tc_tiling_on_sc=False)


@functools.lru_cache(maxsize=None)
def _make_deg(npad, nb):
    rpt = npad // NS

    @functools.partial(
        pl.kernel,
        out_type=jax.ShapeDtypeStruct((NC, npad), jnp.float32),
        mesh=_sc_mesh(),
        compiler_params=_SC_PARAMS,
        scratch_types=[
            pltpu.VMEM((nb, EB), jnp.int32),
            pltpu.VMEM((EB,), jnp.float32),
            pltpu.VMEM((rpt,), jnp.float32),
            pltpu.VMEM_SHARED((npad,), jnp.float32),
        ],
    )
    def deg_kernel(dst_hbm, out_hbm, dst_v, ones_v, zer_v, acc):
        c = lax.axis_index("c")
        s = lax.axis_index("s")
        w = c * NS + s

        @pl.loop(0, EB // L)
        def _(i):
            ones_v[pl.ds(i * L, L)] = jnp.ones((L,), jnp.float32)

        @pl.loop(0, rpt // L)
        def _(i):
            zer_v[pl.ds(i * L, L)] = jnp.zeros((L,), jnp.float32)

        pltpu.sync_copy(zer_v, acc.at[pl.ds(s * rpt, rpt)])
        plsc.subcore_barrier()

        pltpu.sync_copy(dst_hbm.at[w], dst_v)

        @pl.loop(0, nb)
        def _(j):
            pltpu.sync_copy(ones_v, acc.at[dst_v.at[j]], add=True)

        plsc.subcore_barrier()
        pltpu.sync_copy(acc.at[pl.ds(s * rpt, rpt)],
                        out_hbm.at[c, pl.ds(s * rpt, rpt)])

    return deg_kernel


@functools.lru_cache(maxsize=None)
def _make_prop(npad, nb, wd):
    rpt = npad // NS
    stage_y = wd * npad * 8 + 16 * (4 * CH * EB + 8 * EB * wd) <= 7 << 20

    scratch = [
        pltpu.VMEM((2, CH, EB), jnp.int32),
        pltpu.VMEM((2, CH, EB), jnp.int32),
        pltpu.VMEM((2, EB, wd), jnp.float32),
        pltpu.VMEM_SHARED((npad, wd), jnp.float32),
        pltpu.SemaphoreType.DMA,
    ]
    if stage_y:
        scratch.append(pltpu.VMEM_SHARED((npad, wd), jnp.float32))

    @functools.partial(
        pl.kernel,
        out_type=jax.ShapeDtypeStruct((NC, npad, wd), jnp.float32),
        mesh=_sc_mesh(),
        compiler_params=_SC_PARAMS,
        scratch_types=scratch,
    )
    def prop_kernel(y_hbm, src_hbm, dst_hbm, out_hbm,
                    src_v, dst_v, rows_v, acc, sem_g, *maybe_ys):
        c = lax.axis_index("c")
        s = lax.axis_index("s")
        w = c * NS + s

        @pl.loop(0, EB)
        def _(r):
            @pl.loop(0, wd // L)
            def _(q):
                rows_v[0, r, pl.ds(q * L, L)] = jnp.zeros((L,), jnp.float32)

        @pl.loop(0, rpt // EB)
        def _(i):
            pltpu.sync_copy(rows_v.at[0], acc.at[pl.ds(s * rpt + i * EB, EB)])

        if stage_y:
            y_tbl = maybe_ys[0]
            pltpu.sync_copy(y_hbm.at[pl.ds(s * rpt, rpt)],
                            y_tbl.at[pl.ds(s * rpt, rpt)])
        else:
            y_tbl = y_hbm

        plsc.subcore_barrier()

        pltpu.sync_copy(src_hbm.at[w, pl.ds(0, CH)], src_v.at[0])
        pltpu.sync_copy(dst_hbm.at[w, pl.ds(0, CH)], dst_v.at[0])
        pltpu.async_copy(y_tbl.at[src_v.at[0, 0]], rows_v.at[0], sem_g)

        @pl.loop(0, nb)
        def _(t):
            m = lax.div(t, CH)
            mb = lax.rem(m, 2)
            slot = lax.rem(t, CH)
            b = lax.rem(t, 2)

            pltpu.make_async_copy(
                y_tbl.at[src_v.at[mb, slot]], rows_v.at[b], sem_g).wait()

            @pl.when((slot == CH - 1) & (t + 1 < nb))
            def _():
                mb1 = lax.rem(m + 1, 2)
                pltpu.sync_copy(src_hbm.at[w, pl.ds((m + 1) * CH, CH)],
                                src_v.at[mb1])
                pltpu.sync_copy(dst_hbm.at[w, pl.ds((m + 1) * CH, CH)],
                                dst_v.at[mb1])

            @pl.when(t + 1 < nb)
            def _():
                t1 = t + 1
                pltpu.async_copy(
                    y_tbl.at[src_v.at[lax.rem(lax.div(t1, CH), 2),
                                      lax.rem(t1, CH)]],
                    rows_v.at[lax.rem(t1, 2)], sem_g)

            pltpu.sync_copy(rows_v.at[b], acc.at[dst_v.at[mb, slot]],
                            add=True)

        plsc.subcore_barrier()
        pltpu.sync_copy(acc.at[pl.ds(s * rpt, rpt)],
                        out_hbm.at[c, pl.ds(s * rpt, rpt)])

    return prop_kernel


def _tc1(degp, xp, W1):
    npad = xp.shape[0]

    def body(deg_ref, x_ref, w1_ref, y_ref):
        dinv = lax.rsqrt(deg_ref[0] + deg_ref[1] + 1.0)
        y_ref[...] = dinv[:, None] * jnp.dot(
            x_ref[...], w1_ref[...], preferred_element_type=jnp.float32)

    return pl.pallas_call(
        body,
        grid=(npad // RB,),
        in_specs=[
            pl.BlockSpec((NC, RB), lambda i: (0, i)),
            pl.BlockSpec((RB, 128), lambda i: (i, 0)),
            pl.BlockSpec((128, 128), lambda i: (0, 0)),
        ],
        out_specs=pl.BlockSpec((RB, 128), lambda i: (i, 0)),
        out_shape=jax.ShapeDtypeStruct((npad, 128), jnp.float32),
    )(degp, xp, W1)


def _tc2(degp, s1, y1, b1, W2):
    npad = y1.shape[0]
    cd = W2.shape[1]

    def body(deg_ref, s1_ref, y1_ref, b1_ref, w2_ref, y2_ref):
        dinv = lax.rsqrt(deg_ref[0] + deg_ref[1] + 1.0)
        t = s1_ref[0] + s1_ref[1] + y1_ref[...]
        h = jnp.maximum(dinv[:, None] * t + b1_ref[...], 0.0)
        y2_ref[...] = dinv[:, None] * jnp.dot(
            h, w2_ref[...], preferred_element_type=jnp.float32)

    return pl.pallas_call(
        body,
        grid=(npad // RB,),
        in_specs=[
            pl.BlockSpec((NC, RB), lambda i: (0, i)),
            pl.BlockSpec((NC, RB, 128), lambda i: (0, i, 0)),
            pl.BlockSpec((RB, 128), lambda i: (i, 0)),
            pl.BlockSpec((128,), lambda i: (0,)),
            pl.BlockSpec((128, cd), lambda i: (0, 0)),
        ],
        out_specs=pl.BlockSpec((RB, cd), lambda i: (i, 0)),
        out_shape=jax.ShapeDtypeStruct((npad, cd), jnp.float32),
    )(degp, s1, y1, b1, W2)


def _tc3(degp, s2, y2, b2):
    npad = y2.shape[0]
    cd = y2.shape[1]

    def body(deg_ref, s2_ref, y2_ref, b2_ref, o_ref):
        dinv = lax.rsqrt(deg_ref[0] + deg_ref[1] + 1.0)
        o = dinv[:, None] * (s2_ref[0] + s2_ref[1] + y2_ref[...]) + b2_ref[...]
        m = jnp.max(o, axis=-1, keepdims=True)
        lse = jnp.log(jnp.sum(jnp.exp(o - m), axis=-1, keepdims=True)) + m
        o_ref[...] = o - lse

    return pl.pallas_call(
        body,
        grid=(npad // RB,),
        in_specs=[
            pl.BlockSpec((NC, RB), lambda i: (0, i)),
            pl.BlockSpec((NC, RB, cd), lambda i: (0, i, 0)),
            pl.BlockSpec((RB, cd), lambda i: (i, 0)),
            pl.BlockSpec((cd,), lambda i: (0,)),
        ],
        out_specs=pl.BlockSpec((RB, cd), lambda i: (i, 0)),
        out_shape=jax.ShapeDtypeStruct((npad, cd), jnp.float32),
    )(degp, s2, y2, b2)


def kernel(x, edge_index, W1, b1, W2, b2):
    n, d = x.shape
    e = edge_index.shape[1]

    npad = ((n + RB) // RB) * RB
    step = TILES * EB * CH
    epad = ((e + step - 1) // step) * step
    nb = epad // (TILES * EB)

    ei = edge_index.astype(jnp.int32)
    pad = jnp.full((epad - e,), n, jnp.int32)
    src_p = jnp.concatenate([ei[0], pad]).reshape(TILES, nb, EB)
    dst_p = jnp.concatenate([ei[1], pad]).reshape(TILES, nb, EB)
    xp = jnp.pad(x, ((0, npad - n), (0, 0)))

    degp = _make_deg(npad, nb)(dst_p)
    y1 = _tc1(degp, xp, W1)
    prop64 = _make_prop(npad, nb, 64)
    s1a = prop64(y1[:, :64], src_p, dst_p)
    s1b = prop64(y1[:, 64:], src_p, dst_p)
    s1 = jnp.concatenate([s1a, s1b], axis=2)
    y2 = _tc2(degp, s1, y1, b1, W2)
    s2 = _make_prop(npad, nb, 16)(y2, src_p, dst_p)
    out = _tc3(degp, s2, y2, b2)
    return out[:n]
